# Initial kernel scaffold; baseline (speedup 1.0000x reference)
import jax
import jax.numpy as jnp
from jax.experimental import pallas as pl


def kernel(x, tok_embed, pos_embed, ln_gamma, ln_beta):
    raise NotImplementedError("write your pallas kernel here")


# speedup vs baseline: 1.6725x; 1.6725x over previous
import functools

import jax
import jax.numpy as jnp
from jax import lax
from jax.experimental import pallas as pl
from jax.experimental.pallas import tpu as pltpu
from jax.experimental.pallas import tpu_sc as plsc

BATCH = 4
SEQ_LEN = 2048
D_MODEL = 2048
TOKENS = BATCH * SEQ_LEN

NUM_CORES = 2
NUM_SUBCORES = 16
NUM_WORKERS = NUM_CORES * NUM_SUBCORES
ROWS_PER_WORKER = TOKENS // NUM_WORKERS
GATHER_CHUNK = 16

SEQ_BLOCK = 256


def _sc_gather(tok_embed, idx_flat):
    mesh = plsc.VectorSubcoreMesh(core_axis_name="c", subcore_axis_name="s")

    @functools.partial(
        pl.kernel,
        mesh=mesh,
        out_type=jax.ShapeDtypeStruct((TOKENS, D_MODEL), jnp.float32),
        scratch_types=[
            pltpu.VMEM((ROWS_PER_WORKER,), jnp.int32),
            pltpu.VMEM((GATHER_CHUNK, D_MODEL), jnp.float32),
            pltpu.SemaphoreType.DMA,
        ],
    )
    def gather_kernel(table_hbm, idx_hbm, out_hbm, idx_v, rows_v, sem):
        wid = lax.axis_index("s") * NUM_CORES + lax.axis_index("c")
        base = wid * ROWS_PER_WORKER
        pltpu.sync_copy(idx_hbm.at[pl.ds(base, ROWS_PER_WORKER)], idx_v)

        @pl.loop(0, ROWS_PER_WORKER, step=GATHER_CHUNK)
        def _(c):
            pltpu.async_copy(
                table_hbm.at[idx_v.at[pl.ds(c, GATHER_CHUNK)]], rows_v, sem
            ).wait()
            pltpu.sync_copy(rows_v, out_hbm.at[pl.ds(base + c, GATHER_CHUNK)])

    return gather_kernel(tok_embed, idx_flat)


def _ln_body(g_ref, p_ref, gamma_ref, beta_ref, o_ref):
    h = g_ref[...] + p_ref[...]
    mean = jnp.mean(h, axis=1, keepdims=True)
    cent = h - mean
    var = jnp.mean(cent * cent, axis=1, keepdims=True)
    inv = lax.rsqrt(var + 1e-5)
    o_ref[...] = cent * inv * gamma_ref[...] + beta_ref[...]


def _tc_add_layernorm(rows, pos_embed, ln_gamma, ln_beta):
    n_seq_blocks = SEQ_LEN // SEQ_BLOCK
    grid = (n_seq_blocks, BATCH)
    return pl.pallas_call(
        _ln_body,
        grid=grid,
        in_specs=[
            pl.BlockSpec((SEQ_BLOCK, D_MODEL), lambda s, b: (b * (SEQ_LEN // SEQ_BLOCK) + s, 0)),
            pl.BlockSpec((SEQ_BLOCK, D_MODEL), lambda s, b: (s, 0)),
            pl.BlockSpec((1, D_MODEL), lambda s, b: (0, 0)),
            pl.BlockSpec((1, D_MODEL), lambda s, b: (0, 0)),
        ],
        out_specs=pl.BlockSpec(
            (SEQ_BLOCK, D_MODEL), lambda s, b: (b * (SEQ_LEN // SEQ_BLOCK) + s, 0)
        ),
        out_shape=jax.ShapeDtypeStruct((TOKENS, D_MODEL), jnp.float32),
    )(rows, pos_embed, ln_gamma.reshape(1, D_MODEL), ln_beta.reshape(1, D_MODEL))


def kernel(x, tok_embed, pos_embed, ln_gamma, ln_beta):
    idx_flat = x.reshape(TOKENS).astype(jnp.int32)
    rows = _sc_gather(tok_embed, idx_flat)
    out = _tc_add_layernorm(rows, pos_embed, ln_gamma, ln_beta)
    return out.reshape(BATCH, SEQ_LEN, D_MODEL)


# speedup vs baseline: 1.8203x; 1.0884x over previous
import functools

import jax
import jax.numpy as jnp
from jax import lax
from jax.experimental import pallas as pl
from jax.experimental.pallas import tpu as pltpu
from jax.experimental.pallas import tpu_sc as plsc

BATCH = 4
SEQ_LEN = 2048
D_MODEL = 2048
TOKENS = BATCH * SEQ_LEN

NUM_CORES = 2
NUM_SUBCORES = 16
NUM_WORKERS = NUM_CORES * NUM_SUBCORES
ROWS_PER_WORKER = TOKENS // NUM_WORKERS
GATHER_CHUNK = 16

SEQ_BLOCK = 256


def _sc_gather(tok_embed, idx_flat):
    mesh = plsc.VectorSubcoreMesh(core_axis_name="c", subcore_axis_name="s")

    @functools.partial(
        pl.kernel,
        mesh=mesh,
        out_type=jax.ShapeDtypeStruct((TOKENS, D_MODEL), jnp.float32),
        scratch_types=[
            pltpu.VMEM((ROWS_PER_WORKER,), jnp.int32),
            pltpu.VMEM((GATHER_CHUNK, D_MODEL), jnp.float32),
            pltpu.VMEM((GATHER_CHUNK, D_MODEL), jnp.float32),
            pltpu.SemaphoreType.DMA,
            pltpu.SemaphoreType.DMA,
        ],
    )
    def gather_kernel(table_hbm, idx_hbm, out_hbm, idx_v, rows_a, rows_b, sem_a, sem_b):
        wid = lax.axis_index("s") * NUM_CORES + lax.axis_index("c")
        base = wid * ROWS_PER_WORKER
        pltpu.sync_copy(idx_hbm.at[pl.ds(base, ROWS_PER_WORKER)], idx_v)

        n_rows = ROWS_PER_WORKER

        def gather_into(c, buf, sem):
            pltpu.async_copy(
                table_hbm.at[idx_v.at[pl.ds(c, GATHER_CHUNK)]], buf, sem
            )

        def drain(buf, sem):
            pltpu.make_async_copy(
                out_hbm.at[pl.ds(base, GATHER_CHUNK)], buf, sem
            ).wait()

        gather_into(0, rows_a, sem_a)

        @pl.loop(0, n_rows, step=2 * GATHER_CHUNK)
        def _(c):
            gather_into(c + GATHER_CHUNK, rows_b, sem_b)
            drain(rows_a, sem_a)
            pltpu.sync_copy(rows_a, out_hbm.at[pl.ds(base + c, GATHER_CHUNK)])

            nxt = c + 2 * GATHER_CHUNK

            @pl.when(nxt < n_rows)
            def _():
                gather_into(nxt, rows_a, sem_a)

            drain(rows_b, sem_b)
            pltpu.sync_copy(
                rows_b, out_hbm.at[pl.ds(base + c + GATHER_CHUNK, GATHER_CHUNK)]
            )

    return gather_kernel(tok_embed, idx_flat)


def _ln_body(g_ref, p_ref, gamma_ref, beta_ref, o_ref):
    h = g_ref[...] + p_ref[...]
    mean = jnp.mean(h, axis=1, keepdims=True)
    cent = h - mean
    var = jnp.mean(cent * cent, axis=1, keepdims=True)
    inv = lax.rsqrt(var + 1e-5)
    o_ref[...] = cent * inv * gamma_ref[...] + beta_ref[...]


def _tc_add_layernorm(rows, pos_embed, ln_gamma, ln_beta):
    n_seq_blocks = SEQ_LEN // SEQ_BLOCK
    grid = (n_seq_blocks, BATCH)
    return pl.pallas_call(
        _ln_body,
        grid=grid,
        in_specs=[
            pl.BlockSpec((SEQ_BLOCK, D_MODEL), lambda s, b: (b * (SEQ_LEN // SEQ_BLOCK) + s, 0)),
            pl.BlockSpec((SEQ_BLOCK, D_MODEL), lambda s, b: (s, 0)),
            pl.BlockSpec((1, D_MODEL), lambda s, b: (0, 0)),
            pl.BlockSpec((1, D_MODEL), lambda s, b: (0, 0)),
        ],
        out_specs=pl.BlockSpec(
            (SEQ_BLOCK, D_MODEL), lambda s, b: (b * (SEQ_LEN // SEQ_BLOCK) + s, 0)
        ),
        out_shape=jax.ShapeDtypeStruct((TOKENS, D_MODEL), jnp.float32),
    )(rows, pos_embed, ln_gamma.reshape(1, D_MODEL), ln_beta.reshape(1, D_MODEL))


def kernel(x, tok_embed, pos_embed, ln_gamma, ln_beta):
    idx_flat = x.reshape(TOKENS).astype(jnp.int32)
    rows = _sc_gather(tok_embed, idx_flat)
    out = _tc_add_layernorm(rows, pos_embed, ln_gamma, ln_beta)
    return out.reshape(BATCH, SEQ_LEN, D_MODEL)
